# baseline (speedup 1.0000x reference)
import functools

import jax
import jax.numpy as jnp
from jax import lax
from jax.experimental import pallas as pl
from jax.experimental.pallas import tpu as pltpu
from jax.experimental.pallas import tpu_sc as plsc

NUM_CORES = 2
NUM_SUBCORES = 16
NUM_WORKERS = NUM_CORES * NUM_SUBCORES
LANES = 16


def _make_gather(batch, vocab, dim):
    assert batch % NUM_WORKERS == 0
    b_per_w = batch // NUM_WORKERS
    half = vocab // 2
    mesh = plsc.VectorSubcoreMesh(core_axis_name="c", subcore_axis_name="s")

    @functools.partial(
        pl.kernel,
        mesh=mesh,
        out_type=jax.ShapeDtypeStruct((batch, dim), jnp.float32),
        scratch_types=[
            pltpu.VMEM((b_per_w,), jnp.int32),
            pltpu.VMEM((b_per_w,), jnp.int32),
            pltpu.VMEM((b_per_w,), jnp.int32),
            pltpu.VMEM((2 * b_per_w, dim), jnp.float32),
            pltpu.VMEM((b_per_w, dim), jnp.float32),
            pltpu.SemaphoreType.DMA,
        ],
        compiler_params=pltpu.CompilerParams(
            use_tc_tiling_on_sc=False, needs_layout_passes=False),
    )
    def gather_kernel(lo_hbm, hi_hbm, idx_hbm, out_hbm, idx_v, lo_i, hi_i,
                      g_v, sel_v, sem):
        wid = lax.axis_index("s") * NUM_CORES + lax.axis_index("c")
        base = wid * b_per_w
        pltpu.sync_copy(idx_hbm.at[pl.ds(base, b_per_w)], idx_v)
        for g in range(b_per_w // LANES):
            v = idx_v[pl.ds(g * LANES, LANES)]
            in_lo = v < half
            lo_i[pl.ds(g * LANES, LANES)] = jnp.where(in_lo, v, 0)
            hi_i[pl.ds(g * LANES, LANES)] = jnp.where(in_lo, 0, v - half)
        c1 = pltpu.async_copy(lo_hbm.at[lo_i], g_v.at[pl.ds(0, b_per_w)], sem)
        c2 = pltpu.async_copy(hi_hbm.at[hi_i],
                              g_v.at[pl.ds(b_per_w, b_per_w)], sem)
        c1.wait()
        c2.wait()

        def body(g0, carry):
            v = idx_v[pl.ds(g0 * LANES, LANES)]
            rsel = (lax.iota(jnp.int32, LANES) + g0 * LANES
                    + jnp.where(v < half, 0, b_per_w))
            rdst = lax.iota(jnp.int32, LANES) + g0 * LANES
            for c in range(dim):
                csplat = jnp.full((LANES,), c, jnp.int32)
                val = plsc.load_gather(g_v, [rsel, csplat])
                plsc.store_scatter(sel_v, [rdst, csplat], val)
            return carry

        lax.fori_loop(0, b_per_w // LANES, body, 0)
        pltpu.sync_copy(sel_v, out_hbm.at[pl.ds(base, b_per_w)])

    return gather_kernel


@jax.jit
def kernel(indices, weight):
    batch = indices.shape[0]
    vocab, dim = weight.shape
    gather = _make_gather(batch, vocab, dim)
    return gather(weight[: vocab // 2], weight[vocab // 2 :],
                  indices.astype(jnp.int32))


# speedup vs baseline: 1.8363x; 1.8363x over previous
import functools

import jax
import jax.numpy as jnp
from jax import lax
from jax.experimental import pallas as pl
from jax.experimental.pallas import tpu as pltpu
from jax.experimental.pallas import tpu_sc as plsc

NUM_CORES = 2
NUM_SUBCORES = 16
NUM_WORKERS = NUM_CORES * NUM_SUBCORES
LANES = 16


def _make_pair_gather(batch, dim):
    assert batch % (2 * NUM_WORKERS) == 0
    b_per_w = batch // NUM_WORKERS
    q_per_w = b_per_w // 2
    dim2 = 2 * dim
    n_idx_rows = b_per_w // 128
    mesh = plsc.VectorSubcoreMesh(core_axis_name="c", subcore_axis_name="s")

    @functools.partial(
        pl.kernel,
        mesh=mesh,
        out_type=jax.ShapeDtypeStruct((batch // 2, dim2), jnp.float32),
        scratch_types=[
            pltpu.VMEM((b_per_w,), jnp.int32),
            pltpu.VMEM((n_idx_rows, 128), jnp.int32),
            pltpu.VMEM((b_per_w, dim2), jnp.float32),
            pltpu.VMEM((q_per_w, dim2), jnp.float32),
            pltpu.SemaphoreType.DMA,
        ],
        compiler_params=pltpu.CompilerParams(
            use_tc_tiling_on_sc=True, needs_layout_passes=False),
    )
    def pair_gather(table_hbm, idx_hbm, out_hbm, idx_v, pidx_v, pair_v,
                    sel_v, sem):
        wid = lax.axis_index("s") * NUM_CORES + lax.axis_index("c")
        base = wid * b_per_w
        pltpu.sync_copy(idx_hbm.at[pl.ds(base, b_per_w)], idx_v)
        for g in range(b_per_w // LANES):
            v = idx_v[pl.ds(g * LANES, LANES)]
            pidx_v[g // 8, pl.ds((g % 8) * LANES, LANES)] = (
                lax.shift_right_logical(v, 1))
        copies = [
            pltpu.async_copy(
                table_hbm.at[pidx_v.at[j]],
                pair_v.at[pl.ds(j * 128, 128)],
                sem,
            )
            for j in range(n_idx_rows)
        ]
        for c in copies:
            c.wait()

        def body(g, carry):
            vec = idx_v[pl.ds(g * LANES, LANES)]
            hvec = lax.shift_left(vec & 1, 6)
            rvec = lax.iota(jnp.int32, LANES) + g * LANES
            srow = rvec
            scol_base = hvec
            drow = lax.shift_right_logical(rvec, 1)
            dcol_base = lax.shift_left(rvec & 1, 6)
            for c in range(dim):
                off = jnp.full((LANES,), c, jnp.int32)
                val = plsc.load_gather(
                    pair_v, [srow, scol_base + off])
                plsc.store_scatter(
                    sel_v, [drow, dcol_base + off], val)
            return carry

        lax.fori_loop(0, b_per_w // LANES, body, 0)
        pltpu.sync_copy(sel_v, out_hbm.at[pl.ds(wid * q_per_w, q_per_w)])

    return pair_gather


@jax.jit
def kernel(indices, weight):
    batch = indices.shape[0]
    vocab, dim = weight.shape
    table2 = weight.reshape(vocab // 2, 2 * dim)
    gather = _make_pair_gather(batch, dim)
    out2 = gather(table2, indices.astype(jnp.int32))
    return out2.reshape(batch, dim)


# speedup vs baseline: 4.1911x; 2.2824x over previous
import functools

import jax
import jax.numpy as jnp
from jax import lax
from jax.experimental import pallas as pl
from jax.experimental.pallas import tpu as pltpu
from jax.experimental.pallas import tpu_sc as plsc

NUM_CORES = 2
NUM_SUBCORES = 16
NUM_WORKERS = NUM_CORES * NUM_SUBCORES
LANES = 16
CHUNK = 32


def _make_tile_gather(batch, dim):
    assert batch % (2 * NUM_WORKERS) == 0
    b_per_w = batch // NUM_WORKERS
    q_per_w = b_per_w // 2
    n_chunks = b_per_w // CHUNK
    mesh = plsc.VectorSubcoreMesh(core_axis_name="c", subcore_axis_name="s")

    @functools.partial(
        pl.kernel,
        mesh=mesh,
        out_type=jax.ShapeDtypeStruct((batch // 2, 2 * dim), jnp.float32),
        scratch_types=[
            pltpu.VMEM((b_per_w,), jnp.int32),
            pltpu.VMEM((2, CHUNK, 8, dim), jnp.float32),
            pltpu.VMEM((q_per_w, 2 * dim), jnp.float32),
            pltpu.SemaphoreType.DMA,
            pltpu.SemaphoreType.DMA,
        ],
        compiler_params=pltpu.CompilerParams(
            use_tc_tiling_on_sc=True, needs_layout_passes=False),
    )
    def tile_gather(table_hbm, idx_hbm, out_hbm, idx_v, tb_v, sel_v,
                    sem0, sem1):
        wid = lax.axis_index("s") * NUM_CORES + lax.axis_index("c")
        base = wid * b_per_w
        pltpu.sync_copy(idx_hbm.at[pl.ds(base, b_per_w)], idx_v)

        def fire(c, slot, sem):
            for g in range(CHUNK // LANES):
                vec = idx_v[pl.ds(c * CHUNK + g * LANES, LANES)]
                for u in range(LANES):
                    k = g * LANES + u
                    t = lax.shift_right_logical(vec[u], 3)
                    pltpu.async_copy(
                        table_hbm.at[pl.ds(t, 1)],
                        tb_v.at[slot, pl.ds(k, 1)],
                        sem,
                    )

        def drain(sem):
            pltpu.make_async_copy(
                table_hbm.at[pl.ds(0, CHUNK)],
                tb_v.at[0],
                sem,
            ).wait()

        def select(c, slot):
            for g in range(CHUNK // LANES):
                vecv = idx_v[pl.ds(c * CHUNK + g * LANES, LANES)]
                svec = vecv & 7
                kvec = lax.iota(jnp.int32, LANES) + g * LANES
                pvec = lax.shift_left(c * CHUNK + kvec, 6)
                for d in range(dim):
                    val = plsc.load_gather(
                        tb_v.at[slot],
                        [kvec, svec, jnp.full((LANES,), d, jnp.int32)])
                    p = pvec + d
                    plsc.store_scatter(
                        sel_v,
                        [lax.shift_right_logical(p, 7), p & 127],
                        val,
                    )

        fire(0, 0, sem0)

        def chunk_body(h, carry):
            c0 = 2 * h
            fire(c0 + 1, 1, sem1)
            drain(sem0)
            select(c0, 0)

            @pl.when(h + 1 < n_chunks // 2)
            def _():
                fire(c0 + 2, 0, sem0)

            drain(sem1)
            select(c0 + 1, 1)
            return carry

        lax.fori_loop(0, n_chunks // 2, chunk_body, 0)
        pltpu.sync_copy(sel_v, out_hbm.at[pl.ds(wid * q_per_w, q_per_w)])

    return tile_gather


@jax.jit
def kernel(indices, weight):
    batch = indices.shape[0]
    vocab, dim = weight.shape
    table3 = weight.reshape(vocab // 8, 8, dim)
    gather = _make_tile_gather(batch, dim)
    out2 = gather(table3, indices.astype(jnp.int32))
    return out2.reshape(batch, dim)


# speedup vs baseline: 4.5502x; 1.0857x over previous
import functools

import jax
import jax.numpy as jnp
from jax import lax
from jax.experimental import pallas as pl
from jax.experimental.pallas import tpu as pltpu
from jax.experimental.pallas import tpu_sc as plsc

NUM_CORES = 2
NUM_SUBCORES = 16
NUM_WORKERS = NUM_CORES * NUM_SUBCORES
LANES = 16
CHUNK = 32


def _make_tile_gather(batch, dim):
    assert batch % (2 * NUM_WORKERS) == 0
    b_per_w = batch // NUM_WORKERS
    n_chunks = b_per_w // CHUNK
    mesh = plsc.VectorSubcoreMesh(core_axis_name="c", subcore_axis_name="s")

    @functools.partial(
        pl.kernel,
        mesh=mesh,
        out_type=jax.ShapeDtypeStruct((dim, batch), jnp.float32),
        scratch_types=[
            pltpu.VMEM((b_per_w,), jnp.int32),
            pltpu.VMEM((2, CHUNK, 8, dim), jnp.float32),
            pltpu.VMEM((dim, b_per_w), jnp.float32),
            pltpu.SemaphoreType.DMA,
            pltpu.SemaphoreType.DMA,
        ],
        compiler_params=pltpu.CompilerParams(
            use_tc_tiling_on_sc=True, needs_layout_passes=False),
    )
    def tile_gather(table_hbm, idx_hbm, out_hbm, idx_v, tb_v, sel_v,
                    sem0, sem1):
        wid = lax.axis_index("s") * NUM_CORES + lax.axis_index("c")
        base = wid * b_per_w
        pltpu.sync_copy(idx_hbm.at[pl.ds(base, b_per_w)], idx_v)

        def fire(c, slot, sem):
            for g in range(CHUNK // LANES):
                vec = idx_v[pl.ds(c * CHUNK + g * LANES, LANES)]
                for u in range(LANES):
                    k = g * LANES + u
                    t = lax.shift_right_logical(vec[u], 3)
                    pltpu.async_copy(
                        table_hbm.at[pl.ds(t, 1)],
                        tb_v.at[slot, pl.ds(k, 1)],
                        sem,
                    )

        def drain(sem):
            pltpu.make_async_copy(
                table_hbm.at[pl.ds(0, CHUNK)],
                tb_v.at[0],
                sem,
            ).wait()

        def select(c, slot):
            for g in range(CHUNK // LANES):
                vecv = idx_v[pl.ds(c * CHUNK + g * LANES, LANES)]
                svec = vecv & 7
                kvec = lax.iota(jnp.int32, LANES) + g * LANES
                for d in range(dim):
                    val = plsc.load_gather(
                        tb_v.at[slot],
                        [kvec, svec, jnp.full((LANES,), d, jnp.int32)])
                    sel_v[d, pl.ds(c * CHUNK + g * LANES, LANES)] = val

        fire(0, 0, sem0)

        def chunk_body(h, carry):
            c0 = 2 * h
            fire(c0 + 1, 1, sem1)
            drain(sem0)
            select(c0, 0)

            @pl.when(h + 1 < n_chunks // 2)
            def _():
                fire(c0 + 2, 0, sem0)

            drain(sem1)
            select(c0 + 1, 1)
            return carry

        lax.fori_loop(0, n_chunks // 2, chunk_body, 0)
        pltpu.sync_copy(sel_v, out_hbm.at[pl.ds(0, dim), pl.ds(base, b_per_w)])

    return tile_gather


@jax.jit
def kernel(indices, weight):
    batch = indices.shape[0]
    vocab, dim = weight.shape
    table3 = weight.reshape(vocab // 8, 8, dim)
    gather = _make_tile_gather(batch, dim)
    out_t = gather(table3, indices.astype(jnp.int32))
    return out_t.T
